# baseline (speedup 1.0000x reference)
import jax
import jax.numpy as jnp
from jax import lax
from jax.experimental import pallas as pl
from jax.experimental.pallas import tpu as pltpu
from jax.experimental.pallas import tpu_sc as plsc

VOCAB = 1000000
BATCH = 16384
HIST = 50
D = 64

NC = 2
NS = 16
NW = NC * NS

LANES = 16
DSUB = D // LANES

VROWS_PER_W = VOCAB // NW
CP_ROWS = 64
NCP = 488
CP_TAIL = 32
CP_LINES = CP_ROWS // 2

ROWS_PER_W = BATCH // NW
CHUNK_ROWS = 4
IDX_PER_CHUNK = CHUNK_ROWS * HIST
NCHUNK = ROWS_PER_W // CHUNK_ROWS
SHIFT_VECS = (IDX_PER_CHUNK + LANES - 1) // LANES


def _depad_body(table_hbm, t2_hbm, in0, in1, ot0, ot1, si0, si1, so0, so1):
    wid = lax.axis_index("s") * NC + lax.axis_index("c")
    vbase = pl.multiple_of((wid * VROWS_PER_W) // 16 * 16, 16)

    ins = (in0, in1)
    ots = (ot0, ot1)
    isems = (si0, si1)
    osems = (so0, so1)

    def _start_in(cc, p):
        row0 = pl.multiple_of(vbase + cc * CP_ROWS, 16)
        return pltpu.async_copy(table_hbm.at[pl.ds(row0, CP_ROWS)],
                                ins[p], isems[p])

    def _wait_in(p):
        pltpu.make_async_copy(table_hbm.at[pl.ds(0, CP_ROWS)],
                              ins[p], isems[p]).wait()

    def _start_out(cc, p):
        line0 = pl.multiple_of(vbase // 2 + cc * CP_LINES, 8)
        return pltpu.async_copy(ots[p],
                                t2_hbm.at[pl.ds(line0, CP_LINES)], osems[p])

    def _wait_out(p):
        pltpu.make_async_copy(ots[p],
                              t2_hbm.at[pl.ds(0, CP_LINES)], osems[p]).wait()

    def _shuffle(p, nk):
        def _k(k, carry):
            for rr in range(8):
                for j in range(DSUB):
                    off = 64 * rr + 16 * j
                    ots[p][4 * k + off // 128,
                           pl.ds(off % 128, LANES)] = (
                        ins[p][8 * k + rr, pl.ds(16 * j, LANES)])
            return carry
        lax.fori_loop(0, nk, _k, 0, unroll=False)

    _start_in(0, 0)
    _start_in(1, 1)
    _wait_in(0)
    _shuffle(0, CP_ROWS // 8)
    _start_out(0, 0)
    _start_in(2, 0)
    _wait_in(1)
    _shuffle(1, CP_ROWS // 8)
    _start_out(1, 1)
    _start_in(3, 1)

    def _step(i, carry):
        for p in range(2):
            cc = 2 * i + 2 + p
            _wait_out(p)
            _wait_in(p)
            _shuffle(p, CP_ROWS // 8)
            _start_out(cc, p)

            @pl.when(cc + 2 < NCP)
            def _next(p=p, cc=cc):
                _start_in(cc + 2, p)
        return carry

    lax.fori_loop(0, (NCP - 2) // 2, _step, 0, unroll=False)

    _wait_out(0)
    tail0 = pl.multiple_of(vbase + NCP * CP_ROWS, 16)
    pltpu.sync_copy(table_hbm.at[pl.ds(tail0, CP_TAIL)],
                    in0.at[pl.ds(0, CP_TAIL)])
    _shuffle(0, CP_TAIL // 8)
    pltpu.sync_copy(ot0.at[pl.ds(0, CP_TAIL // 2)],
                    t2_hbm.at[pl.ds(pl.multiple_of(tail0 // 2, 8),
                                    CP_TAIL // 2)])
    _wait_out(1)


def _depad(table):
    mesh = plsc.VectorSubcoreMesh(core_axis_name="c", subcore_axis_name="s",
                                  num_cores=NC, num_subcores=NS)
    fn = pl.kernel(
        _depad_body,
        out_type=jax.ShapeDtypeStruct((VOCAB // 2, 2 * D), jnp.float32),
        mesh=mesh,
        scratch_types=[
            pltpu.VMEM((CP_ROWS, D), jnp.float32),
            pltpu.VMEM((CP_ROWS, D), jnp.float32),
            pltpu.VMEM((CP_LINES, 2 * D), jnp.float32),
            pltpu.VMEM((CP_LINES, 2 * D), jnp.float32),
            pltpu.SemaphoreType.DMA,
            pltpu.SemaphoreType.DMA,
            pltpu.SemaphoreType.DMA,
            pltpu.SemaphoreType.DMA,
        ],
    )
    return fn(table)


def _sc_body(actions_hbm, t2_hbm, out_hbm,
             idx_v, sidx0, sidx1, buf0, buf1, out_v, sem0, sem1):
    wid = lax.axis_index("s") * NC + lax.axis_index("c")
    base_idx = wid * ROWS_PER_W * HIST

    pltpu.sync_copy(actions_hbm.at[pl.ds(pl.multiple_of(base_idx, 8),
                                         ROWS_PER_W * HIST)],
                    idx_v.at[pl.ds(0, ROWS_PER_W * HIST)])

    sidxs = (sidx0, sidx1)
    bufs = (buf0, buf1)
    sems = (sem0, sem1)

    def _start_gather(chunk, sidx, buf, sem):
        for k in range(SHIFT_VECS):
            off = chunk * IDX_PER_CHUNK + k * LANES
            sidx[pl.ds(k * LANES, LANES)] = (
                lax.shift_right_logical(idx_v[pl.ds(off, LANES)], 1))
        return pltpu.async_copy(
            t2_hbm.at[sidx.at[pl.ds(0, IDX_PER_CHUNK)]], buf, sem)

    _start_gather(0, sidx0, buf0, sem0)
    _start_gather(1, sidx1, buf1, sem1)

    def _reduce_chunk(chunk, buf):
        def _row(r, carry):
            e0 = (chunk * CHUNK_ROWS + r) * HIST
            iv = [idx_v[pl.ds(e0, LANES)],
                  idx_v[pl.ds(e0 + 16, LANES)],
                  idx_v[pl.ds(e0 + 32, LANES)],
                  idx_v[pl.ds(e0 + 34, LANES)]]

            def col(i):
                a = iv[i // 16][i % 16] if i < 48 else iv[3][i - 34]
                return (a & 1) * D

            acc = [buf[r * HIST, pl.ds(col(0) + j * LANES, LANES)]
                   for j in range(DSUB)]
            for i in range(1, HIST):
                ci = col(i)
                for j in range(DSUB):
                    acc[j] = acc[j] + buf[r * HIST + i,
                                          pl.ds(ci + j * LANES, LANES)]
            obase = (chunk * CHUNK_ROWS + r) * D
            for j in range(DSUB):
                out_v[pl.ds(obase + j * LANES, LANES)] = acc[j]
            return carry
        lax.fori_loop(0, CHUNK_ROWS, _row, 0, unroll=False)

    def _step(i, carry):
        for p in range(2):
            chunk = 2 * i + p
            pltpu.make_async_copy(
                t2_hbm.at[sidxs[p].at[pl.ds(0, IDX_PER_CHUNK)]],
                bufs[p], sems[p]).wait()
            _reduce_chunk(chunk, bufs[p])

            @pl.when(i < NCHUNK // 2 - 1)
            def _start_next(p=p, chunk=chunk):
                _start_gather(chunk + 2, sidxs[p], bufs[p], sems[p])
        return carry

    lax.fori_loop(0, NCHUNK // 2, _step, 0, unroll=False)

    pltpu.sync_copy(out_v,
                    out_hbm.at[pl.ds(pl.multiple_of(wid * ROWS_PER_W * D, 8),
                                     ROWS_PER_W * D)])


def _sc_gather_pool(actions_flat, t2):
    mesh = plsc.VectorSubcoreMesh(core_axis_name="c", subcore_axis_name="s",
                                  num_cores=NC, num_subcores=NS)
    fn = pl.kernel(
        _sc_body,
        out_type=jax.ShapeDtypeStruct((BATCH * D,), jnp.float32),
        mesh=mesh,
        scratch_types=[
            pltpu.VMEM((ROWS_PER_W * HIST + LANES,), jnp.int32),
            pltpu.VMEM((SHIFT_VECS * LANES,), jnp.int32),
            pltpu.VMEM((SHIFT_VECS * LANES,), jnp.int32),
            pltpu.VMEM((IDX_PER_CHUNK, 2 * D), jnp.float32),
            pltpu.VMEM((IDX_PER_CHUNK, 2 * D), jnp.float32),
            pltpu.VMEM((ROWS_PER_W * D,), jnp.float32),
            pltpu.SemaphoreType.DMA,
            pltpu.SemaphoreType.DMA,
        ],
    )
    return fn(actions_flat, t2)


def _tc_project_body(x_ref, w_ref, b_ref, o_ref):
    x = x_ref[...] * (1.0 / HIST)
    o_ref[...] = lax.dot_general(
        x, w_ref[...], (((1,), (1,)), ((), ())),
        preferred_element_type=jnp.float32) + b_ref[...]


def _tc_project(pooled, w, b2):
    bm = 1024
    return pl.pallas_call(
        _tc_project_body,
        grid=(BATCH // bm,),
        in_specs=[
            pl.BlockSpec((bm, D), lambda i: (i, 0)),
            pl.BlockSpec((D, D), lambda i: (0, 0)),
            pl.BlockSpec((1, D), lambda i: (0, 0)),
        ],
        out_specs=pl.BlockSpec((bm, D), lambda i: (i, 0)),
        out_shape=jax.ShapeDtypeStruct((BATCH, D), jnp.float32),
    )(pooled, w, b2)


def kernel(actions, emb_table, W, b):
    actions_flat = actions.reshape(-1).astype(jnp.int32)
    t2 = _depad(emb_table)
    pooled = _sc_gather_pool(actions_flat, t2).reshape(BATCH, D)
    return _tc_project(pooled, W, b.reshape(1, D))


# speedup vs baseline: 1.2903x; 1.2903x over previous
import jax
import jax.numpy as jnp
from jax import lax
from jax.experimental import pallas as pl
from jax.experimental.pallas import tpu as pltpu
from jax.experimental.pallas import tpu_sc as plsc

VOCAB = 1000000
BATCH = 16384
HIST = 50
D = 64

NC = 2
NS = 16
NW = NC * NS

LANES = 16
DSUB = D // LANES

VROWS_PER_W = VOCAB // NW
CP_ROWS = 64
NCP = 488
CP_TAIL = 32
CP_LINES = CP_ROWS // 2

ROWS_PER_W = BATCH // NW
CHUNK_ROWS = 4
IDX_PER_CHUNK = CHUNK_ROWS * HIST
NCHUNK = ROWS_PER_W // CHUNK_ROWS
SHIFT_VECS = (IDX_PER_CHUNK + LANES - 1) // LANES


def _sc_body(actions_hbm, t2_hbm, out_hbm,
             idx_v, sidx0, sidx1, buf0, buf1, out_v, sem0, sem1):
    wid = lax.axis_index("s") * NC + lax.axis_index("c")
    base_idx = wid * ROWS_PER_W * HIST

    pltpu.sync_copy(actions_hbm.at[pl.ds(pl.multiple_of(base_idx, 8),
                                         ROWS_PER_W * HIST)],
                    idx_v.at[pl.ds(0, ROWS_PER_W * HIST)])

    sidxs = (sidx0, sidx1)
    bufs = (buf0, buf1)
    sems = (sem0, sem1)

    def _start_gather(chunk, sidx, buf, sem):
        for k in range(SHIFT_VECS):
            off = chunk * IDX_PER_CHUNK + k * LANES
            sidx[pl.ds(k * LANES, LANES)] = (
                lax.shift_right_logical(idx_v[pl.ds(off, LANES)], 1))
        return pltpu.async_copy(
            t2_hbm.at[sidx.at[pl.ds(0, IDX_PER_CHUNK)]], buf, sem)

    _start_gather(0, sidx0, buf0, sem0)
    _start_gather(1, sidx1, buf1, sem1)

    def _reduce_chunk(chunk, buf):
        def _row(r, carry):
            e0 = (chunk * CHUNK_ROWS + r) * HIST
            iv = [idx_v[pl.ds(e0, LANES)],
                  idx_v[pl.ds(e0 + 16, LANES)],
                  idx_v[pl.ds(e0 + 32, LANES)],
                  idx_v[pl.ds(e0 + 34, LANES)]]

            def col(i):
                a = iv[i // 16][i % 16] if i < 48 else iv[3][i - 34]
                return (a & 1) * D

            acc = [buf[r * HIST, pl.ds(col(0) + j * LANES, LANES)]
                   for j in range(DSUB)]
            for i in range(1, HIST):
                ci = col(i)
                for j in range(DSUB):
                    acc[j] = acc[j] + buf[r * HIST + i,
                                          pl.ds(ci + j * LANES, LANES)]
            obase = (chunk * CHUNK_ROWS + r) * D
            for j in range(DSUB):
                out_v[pl.ds(obase + j * LANES, LANES)] = acc[j]
            return carry
        lax.fori_loop(0, CHUNK_ROWS, _row, 0, unroll=False)

    def _step(i, carry):
        for p in range(2):
            chunk = 2 * i + p
            pltpu.make_async_copy(
                t2_hbm.at[sidxs[p].at[pl.ds(0, IDX_PER_CHUNK)]],
                bufs[p], sems[p]).wait()
            _reduce_chunk(chunk, bufs[p])

            @pl.when(i < NCHUNK // 2 - 1)
            def _start_next(p=p, chunk=chunk):
                _start_gather(chunk + 2, sidxs[p], bufs[p], sems[p])
        return carry

    lax.fori_loop(0, NCHUNK // 2, _step, 0, unroll=False)

    pltpu.sync_copy(out_v,
                    out_hbm.at[pl.ds(pl.multiple_of(wid * ROWS_PER_W * D, 8),
                                     ROWS_PER_W * D)])


def _sc_gather_pool(actions_flat, t2):
    mesh = plsc.VectorSubcoreMesh(core_axis_name="c", subcore_axis_name="s",
                                  num_cores=NC, num_subcores=NS)
    fn = pl.kernel(
        _sc_body,
        out_type=jax.ShapeDtypeStruct((BATCH * D,), jnp.float32),
        mesh=mesh,
        compiler_params=pltpu.CompilerParams(use_tc_tiling_on_sc=False),
        scratch_types=[
            pltpu.VMEM((ROWS_PER_W * HIST + LANES,), jnp.int32),
            pltpu.VMEM((SHIFT_VECS * LANES,), jnp.int32),
            pltpu.VMEM((SHIFT_VECS * LANES,), jnp.int32),
            pltpu.VMEM((IDX_PER_CHUNK, 2 * D), jnp.float32),
            pltpu.VMEM((IDX_PER_CHUNK, 2 * D), jnp.float32),
            pltpu.VMEM((ROWS_PER_W * D,), jnp.float32),
            pltpu.SemaphoreType.DMA,
            pltpu.SemaphoreType.DMA,
        ],
    )
    return fn(actions_flat, t2)


def _tc_project_body(x_ref, w_ref, b_ref, o_ref):
    x = x_ref[...] * (1.0 / HIST)
    o_ref[...] = lax.dot_general(
        x, w_ref[...], (((1,), (1,)), ((), ())),
        preferred_element_type=jnp.float32) + b_ref[...]


def _tc_project(pooled, w, b2):
    bm = 1024
    return pl.pallas_call(
        _tc_project_body,
        grid=(BATCH // bm,),
        in_specs=[
            pl.BlockSpec((bm, D), lambda i: (i, 0)),
            pl.BlockSpec((D, D), lambda i: (0, 0)),
            pl.BlockSpec((1, D), lambda i: (0, 0)),
        ],
        out_specs=pl.BlockSpec((bm, D), lambda i: (i, 0)),
        out_shape=jax.ShapeDtypeStruct((BATCH, D), jnp.float32),
    )(pooled, w, b2)


def kernel(actions, emb_table, W, b):
    actions_flat = actions.reshape(-1).astype(jnp.int32)
    t2 = emb_table.reshape(VOCAB // 2, 2 * D)
    pooled = _sc_gather_pool(actions_flat, t2).reshape(BATCH, D)
    return _tc_project(pooled, W, b.reshape(1, D))


# speedup vs baseline: 1.4327x; 1.1104x over previous
import functools

import jax
import jax.numpy as jnp
from jax import lax
from jax.experimental import pallas as pl
from jax.experimental.pallas import tpu as pltpu
from jax.experimental.pallas import tpu_sc as plsc

BATCH = 16384
HIST = 50
D = 64

NC = 2
NS = 16
NW = NC * NS

ROWS_PER_W = BATCH // NW
CHUNK_ROWS = 8
IDX_PER_CHUNK = CHUNK_ROWS * HIST
NCHUNK = ROWS_PER_W // CHUNK_ROWS
LANES = 16
DSUB = D // LANES


def _sc_body(actions_hbm, table_hbm, out_hbm,
             idx_v, buf0, buf1, out_v, sem0, sem1):
    wid = lax.axis_index("s") * NC + lax.axis_index("c")
    base_row = wid * ROWS_PER_W
    base_idx = base_row * HIST

    pltpu.sync_copy(actions_hbm.at[pl.ds(pl.multiple_of(base_idx, 8),
                                         ROWS_PER_W * HIST)], idx_v)

    bufs = (buf0, buf1)
    sems = (sem0, sem1)

    def _start_gather(chunk, buf, sem):
        off = pl.multiple_of(chunk * IDX_PER_CHUNK, 8)
        return pltpu.async_copy(
            table_hbm.at[idx_v.at[pl.ds(off, IDX_PER_CHUNK)]], buf, sem)

    _start_gather(0, buf0, sem0)
    _start_gather(1, buf1, sem1)

    def _reduce_chunk(chunk, buf):
        def _row(r, _):
            row0 = r * HIST
            acc = [buf[row0, pl.ds(j * LANES, LANES)] for j in range(DSUB)]
            for i in range(1, HIST):
                for j in range(DSUB):
                    acc[j] = acc[j] + buf[row0 + i, pl.ds(j * LANES, LANES)]
            orow = chunk * CHUNK_ROWS + r
            for j in range(DSUB):
                out_v[orow, pl.ds(j * LANES, LANES)] = acc[j]
            return _
        lax.fori_loop(0, CHUNK_ROWS, _row, 0, unroll=False)

    def _step(i, carry):
        for p in range(2):
            chunk = 2 * i + p
            pltpu.make_async_copy(
                table_hbm.at[idx_v.at[pl.ds(0, IDX_PER_CHUNK)]],
                bufs[p], sems[p]).wait()
            _reduce_chunk(chunk, bufs[p])

            @pl.when(i < NCHUNK // 2 - 1)
            def _start_next(p=p, chunk=chunk):
                _start_gather(chunk + 2, bufs[p], sems[p])
        return carry

    lax.fori_loop(0, NCHUNK // 2, _step, 0, unroll=False)

    pltpu.sync_copy(out_v,
                    out_hbm.at[pl.ds(pl.multiple_of(base_row, 8),
                                     ROWS_PER_W)])


@functools.partial(jax.jit, static_argnums=())
def _sc_gather_pool(actions_flat, table):
    mesh = plsc.VectorSubcoreMesh(core_axis_name="c", subcore_axis_name="s",
                                  num_cores=NC, num_subcores=NS)
    fn = pl.kernel(
        _sc_body,
        out_type=jax.ShapeDtypeStruct((BATCH, D), jnp.float32),
        mesh=mesh,
        compiler_params=pltpu.CompilerParams(use_tc_tiling_on_sc=False),
        scratch_types=[
            pltpu.VMEM((ROWS_PER_W * HIST,), jnp.int32),
            pltpu.VMEM((IDX_PER_CHUNK, D), jnp.float32),
            pltpu.VMEM((IDX_PER_CHUNK, D), jnp.float32),
            pltpu.VMEM((ROWS_PER_W, D), jnp.float32),
            pltpu.SemaphoreType.DMA,
            pltpu.SemaphoreType.DMA,
        ],
    )
    return fn(actions_flat, table)


def _tc_project_body(x_ref, w_ref, b_ref, o_ref):
    x = x_ref[...] * (1.0 / HIST)
    o_ref[...] = lax.dot_general(
        x, w_ref[...], (((1,), (1,)), ((), ())),
        preferred_element_type=jnp.float32) + b_ref[...]


def _tc_project(pooled, w, b2):
    bm = 1024
    return pl.pallas_call(
        _tc_project_body,
        grid=(BATCH // bm,),
        in_specs=[
            pl.BlockSpec((bm, D), lambda i: (i, 0)),
            pl.BlockSpec((D, D), lambda i: (0, 0)),
            pl.BlockSpec((1, D), lambda i: (0, 0)),
        ],
        out_specs=pl.BlockSpec((bm, D), lambda i: (i, 0)),
        out_shape=jax.ShapeDtypeStruct((BATCH, D), jnp.float32),
    )(pooled, w, b2)


def kernel(actions, emb_table, W, b):
    actions_flat = actions.reshape(-1).astype(jnp.int32)
    pooled = _sc_gather_pool(actions_flat, emb_table)
    return _tc_project(pooled, W, b.reshape(1, D))
